# baseline (speedup 1.0000x reference)
import functools

import jax
import jax.numpy as jnp
from jax import lax
from jax.experimental import pallas as pl
from jax.experimental.pallas import tpu as pltpu
from jax.experimental.pallas import tpu_sc as plsc

NUM_T = 1000
BATCH = 16384
L = 16
NC = 2
NS = 16
B_PER_W = BATCH // NS
CHUNKS = B_PER_W // L


def _ddpm_lookup(t, beta, alpha):
    mesh = plsc.VectorSubcoreMesh(core_axis_name="c", subcore_axis_name="s")

    @functools.partial(
        pl.kernel,
        mesh=mesh,
        out_type=(
            jax.ShapeDtypeStruct((BATCH,), jnp.float32),
            jax.ShapeDtypeStruct((BATCH,), jnp.float32),
        ),
        scratch_types=[
            pltpu.VMEM((NUM_T,), jnp.float32),
            pltpu.VMEM((B_PER_W,), jnp.int32),
            pltpu.VMEM((B_PER_W,), jnp.float32),
            pltpu.SemaphoreType.DMA,
            pltpu.SemaphoreType.DMA,
        ],
        compiler_params=pltpu.CompilerParams(needs_layout_passes=False),
    )
    def k(t_hbm, beta_hbm, alpha_hbm, beta_out, alpha_out,
          tab_v, idx_v, o_v, in_sem, out_sem):
        cid = lax.axis_index("c")
        base = lax.axis_index("s") * B_PER_W

        def run(table_hbm, out_hbm):
            c1 = pltpu.async_copy(table_hbm, tab_v, in_sem)
            c2 = pltpu.async_copy(t_hbm.at[pl.ds(base, B_PER_W)], idx_v, in_sem)
            c1.wait()
            c2.wait()

            half = B_PER_W // 2

            @plsc.parallel_loop(0, half, step=L, unroll=4)
            def body_lo(off):
                idx = idx_v[pl.ds(off, L)]
                o_v[pl.ds(off, L)] = plsc.load_gather(tab_v, [idx])

            o1 = pltpu.async_copy(
                o_v.at[pl.ds(0, half)], out_hbm.at[pl.ds(base, half)], out_sem)

            @plsc.parallel_loop(half, B_PER_W, step=L, unroll=4)
            def body_hi(off):
                idx = idx_v[pl.ds(off, L)]
                o_v[pl.ds(off, L)] = plsc.load_gather(tab_v, [idx])

            o2 = pltpu.async_copy(
                o_v.at[pl.ds(half, half)],
                out_hbm.at[pl.ds(base + half, half)], out_sem)
            o1.wait()
            o2.wait()

        @pl.when(cid == 0)
        def _():
            run(beta_hbm, beta_out)

        @pl.when(cid == 1)
        def _():
            run(alpha_hbm, alpha_out)

    return k(t, beta, alpha)


def kernel(t, beta, alpha):
    beta_t, alpha_t = _ddpm_lookup(t, beta, alpha)
    return (beta_t, alpha_t)


# speedup vs baseline: 1.0135x; 1.0135x over previous
import functools

import jax
import jax.numpy as jnp
from jax import lax
from jax.experimental import pallas as pl
from jax.experimental.pallas import tpu as pltpu
from jax.experimental.pallas import tpu_sc as plsc

NUM_T = 1000
BATCH = 16384
L = 16
NC = 2
NS = 16
B_PER_W = BATCH // NS
CHUNKS = B_PER_W // L


def _ddpm_lookup(t, beta, alpha):
    mesh = plsc.VectorSubcoreMesh(core_axis_name="c", subcore_axis_name="s")

    @functools.partial(
        pl.kernel,
        mesh=mesh,
        out_type=(
            jax.ShapeDtypeStruct((BATCH,), jnp.float32),
            jax.ShapeDtypeStruct((BATCH,), jnp.float32),
        ),
        scratch_types=[
            pltpu.VMEM((NUM_T,), jnp.float32),
            pltpu.VMEM((B_PER_W,), jnp.int32),
            pltpu.VMEM((B_PER_W,), jnp.float32),
            pltpu.SemaphoreType.DMA,
            pltpu.SemaphoreType.DMA,
        ],
        compiler_params=pltpu.CompilerParams(needs_layout_passes=False),
    )
    def k(t_hbm, beta_hbm, alpha_hbm, beta_out, alpha_out,
          tab_v, idx_v, o_v, in_sem, out_sem):
        cid = lax.axis_index("c")
        base = lax.axis_index("s") * B_PER_W

        def run(table_hbm, out_hbm):
            c1 = pltpu.async_copy(table_hbm, tab_v, in_sem)
            c2 = pltpu.async_copy(t_hbm.at[pl.ds(base, B_PER_W)], idx_v, in_sem)
            c1.wait()
            c2.wait()

            @plsc.parallel_loop(0, B_PER_W, step=L, unroll=4)
            def body(off):
                idx = idx_v[pl.ds(off, L)]
                o_v[pl.ds(off, L)] = plsc.load_gather(tab_v, [idx])

            pltpu.async_copy(o_v, out_hbm.at[pl.ds(base, B_PER_W)], out_sem).wait()

        @pl.when(cid == 0)
        def _():
            run(beta_hbm, beta_out)

        @pl.when(cid == 1)
        def _():
            run(alpha_hbm, alpha_out)

    return k(t, beta, alpha)


def kernel(t, beta, alpha):
    beta_t, alpha_t = _ddpm_lookup(t, beta, alpha)
    return (beta_t, alpha_t)
